# Initial kernel scaffold; baseline (speedup 1.0000x reference)
import jax
import jax.numpy as jnp
from jax.experimental import pallas as pl


def kernel(x, edge_index, batch, W1, b1, g1, be1, W2, b2, g2, be2, W3, b3, g3, be3, Wl, bl):
    raise NotImplementedError("write your pallas kernel here")


# speedup vs baseline: 16.5563x; 16.5563x over previous
import functools

import jax
import jax.numpy as jnp
from jax import lax
from jax.experimental import pallas as pl
from jax.experimental.pallas import tpu as pltpu
from jax.experimental.pallas import tpu_sc as plsc

N_NODES = 10000
N_EDGES = 320000
D = 128
N_GRAPHS = 64
N_CLASSES = 10

NCORE = 2
NSUB = 16
NW = NCORE * NSUB
EPW = N_EDGES // NW
CH = 80
NCH = EPW // CH
NPAD = 10240
ROWS_PER_SUB = NPAD // NSUB
DEGW = 8

_mesh = plsc.VectorSubcoreMesh(core_axis_name="c", subcore_axis_name="s")


@functools.partial(
    pl.kernel,
    out_type=jax.ShapeDtypeStruct((NCORE, NPAD, DEGW), jnp.float32),
    mesh=_mesh,
    scratch_types=[
        pltpu.VMEM((NCH, CH), jnp.int32),
        pltpu.VMEM((CH, DEGW), jnp.float32),
        pltpu.VMEM_SHARED((NPAD, DEGW), jnp.float32),
    ],
    compiler_params=pltpu.CompilerParams(use_tc_tiling_on_sc=False),
)
def _deg(dst_hbm, ones_hbm, zeros_hbm, out_hbm, dst_v, ones_v, deg_sh):
    c = lax.axis_index("c")
    s = lax.axis_index("s")
    wid = c * NSUB + s
    pltpu.sync_copy(dst_hbm.at[wid], dst_v)
    pltpu.sync_copy(ones_hbm, ones_v)
    pltpu.sync_copy(zeros_hbm, deg_sh.at[pl.ds(s * ROWS_PER_SUB, ROWS_PER_SUB)])
    plsc.subcore_barrier()

    def body(j, carry):
        pltpu.sync_copy(ones_v, deg_sh.at[dst_v.at[j]], add=True)
        return carry

    lax.fori_loop(0, NCH, body, 0)
    plsc.subcore_barrier()
    pltpu.sync_copy(deg_sh.at[pl.ds(s * ROWS_PER_SUB, ROWS_PER_SUB)],
                    out_hbm.at[c, pl.ds(s * ROWS_PER_SUB, ROWS_PER_SUB)])


@functools.partial(
    pl.kernel,
    out_type=jax.ShapeDtypeStruct((NCORE, NPAD, D), jnp.float32),
    mesh=_mesh,
    scratch_types=[
        pltpu.VMEM((NCH, CH), jnp.int32),
        pltpu.VMEM((NCH, CH), jnp.int32),
        pltpu.VMEM((CH, D), jnp.float32),
        pltpu.VMEM_SHARED((NPAD, D), jnp.float32),
        pltpu.SemaphoreType.DMA,
    ],
)
def _prop(g_hbm, src_hbm, dst_hbm, zeros_hbm, out_hbm,
          src_v, dst_v, rows_v, acc_sh, sem):
    c = lax.axis_index("c")
    s = lax.axis_index("s")
    wid = c * NSUB + s
    pltpu.sync_copy(src_hbm.at[wid], src_v)
    pltpu.sync_copy(dst_hbm.at[wid], dst_v)
    pltpu.sync_copy(zeros_hbm, acc_sh.at[pl.ds(s * ROWS_PER_SUB, ROWS_PER_SUB)])
    plsc.subcore_barrier()

    def body(j, carry):
        pltpu.async_copy(g_hbm.at[src_v.at[j]], rows_v, sem).wait()
        pltpu.sync_copy(rows_v, acc_sh.at[dst_v.at[j]], add=True)
        return carry

    lax.fori_loop(0, NCH, body, 0)
    plsc.subcore_barrier()
    pltpu.sync_copy(acc_sh.at[pl.ds(s * ROWS_PER_SUB, ROWS_PER_SUB)],
                    out_hbm.at[c, pl.ds(s * ROWS_PER_SUB, ROWS_PER_SUB)])


def _pre_body(x_ref, w_ref, deg_ref, g_out, dis_out):
    deg = deg_ref[0, 0:N_NODES, 0:1] + deg_ref[1, 0:N_NODES, 0:1] + 1.0
    dis = lax.rsqrt(deg)
    h = jnp.dot(x_ref[...], w_ref[...],
                preferred_element_type=jnp.float32,
                precision=lax.Precision.HIGHEST)
    g_out[...] = h * dis
    dis_out[...] = dis


def _mid_body(acc_ref, g_ref, dis_ref, b_ref, gam_ref, bet_ref, w_ref, gout_ref):
    dis = dis_ref[...]
    sres = ((acc_ref[0, 0:N_NODES, :] + acc_ref[1, 0:N_NODES, :]
            + g_ref[...]) * dis + b_ref[...])
    mu = jnp.mean(sres, axis=0, keepdims=True)
    var = jnp.mean((sres - mu) ** 2, axis=0, keepdims=True)
    h = jnp.maximum((sres - mu) * lax.rsqrt(var + 1e-5) * gam_ref[...]
                    + bet_ref[...], 0.0)
    gout_ref[...] = jnp.dot(h, w_ref[...],
                            preferred_element_type=jnp.float32,
                            precision=lax.Precision.HIGHEST) * dis


def _fin_body(acc_ref, g_ref, dis_ref, b_ref, gam_ref, bet_ref,
              batch_ref, wl_ref, bl_ref, out_ref):
    dis = dis_ref[...]
    sres = ((acc_ref[0, 0:N_NODES, :] + acc_ref[1, 0:N_NODES, :]
            + g_ref[...]) * dis + b_ref[...])
    mu = jnp.mean(sres, axis=0, keepdims=True)
    var = jnp.mean((sres - mu) ** 2, axis=0, keepdims=True)
    h = jnp.maximum((sres - mu) * lax.rsqrt(var + 1e-5) * gam_ref[...]
                    + bet_ref[...], 0.0)
    gids = lax.broadcasted_iota(jnp.int32, (N_GRAPHS, N_NODES), 0)
    mask = jnp.where(gids == batch_ref[...], 1.0, 0.0)
    pooled_sum = jnp.dot(mask, h, preferred_element_type=jnp.float32,
                         precision=lax.Precision.HIGHEST)
    cnt = jnp.sum(mask, axis=1, keepdims=True)
    pooled = pooled_sum / jnp.maximum(cnt, 1.0)
    out_ref[...] = jnp.dot(pooled, wl_ref[...],
                           preferred_element_type=jnp.float32,
                           precision=lax.Precision.HIGHEST) + bl_ref[...]


_pre = pl.pallas_call(
    _pre_body,
    out_shape=[jax.ShapeDtypeStruct((N_NODES, D), jnp.float32),
               jax.ShapeDtypeStruct((N_NODES, 1), jnp.float32)],
)

_mid = pl.pallas_call(
    _mid_body,
    out_shape=jax.ShapeDtypeStruct((N_NODES, D), jnp.float32),
)

_fin = pl.pallas_call(
    _fin_body,
    out_shape=jax.ShapeDtypeStruct((N_GRAPHS, N_CLASSES), jnp.float32),
)


def kernel(x, edge_index, batch, W1, b1, g1, be1, W2, b2, g2, be2,
           W3, b3, g3, be3, Wl, bl):
    src = edge_index[0].astype(jnp.int32).reshape(NW, NCH, CH)
    dst = edge_index[1].astype(jnp.int32).reshape(NW, NCH, CH)
    ones_deg = jnp.ones((CH, DEGW), jnp.float32)
    zeros_deg = jnp.zeros((ROWS_PER_SUB, DEGW), jnp.float32)
    zeros_acc = jnp.zeros((ROWS_PER_SUB, D), jnp.float32)
    b1r, b2r, b3r = b1.reshape(1, D), b2.reshape(1, D), b3.reshape(1, D)
    g1r, g2r, g3r = g1.reshape(1, D), g2.reshape(1, D), g3.reshape(1, D)
    be1r, be2r, be3r = be1.reshape(1, D), be2.reshape(1, D), be3.reshape(1, D)
    batch_r = batch.astype(jnp.int32).reshape(1, N_NODES)

    deg2 = _deg(dst, ones_deg, zeros_deg)
    gl1, dis = _pre(x, W1, deg2)
    acc1 = _prop(gl1, src, dst, zeros_acc)
    gl2 = _mid(acc1, gl1, dis, b1r, g1r, be1r, W2)
    acc2 = _prop(gl2, src, dst, zeros_acc)
    gl3 = _mid(acc2, gl2, dis, b2r, g2r, be2r, W3)
    acc3 = _prop(gl3, src, dst, zeros_acc)
    return _fin(acc3, gl3, dis, b3r, g3r, be3r, batch_r, Wl,
                bl.reshape(1, N_CLASSES))


# speedup vs baseline: 20.7528x; 1.2535x over previous
import functools

import jax
import jax.numpy as jnp
from jax import lax
from jax.experimental import pallas as pl
from jax.experimental.pallas import tpu as pltpu
from jax.experimental.pallas import tpu_sc as plsc

N_NODES = 10000
N_EDGES = 320000
D = 128
N_GRAPHS = 64
N_CLASSES = 10

NCORE = 2
NSUB = 16
NW = NCORE * NSUB
EPW = N_EDGES // NW
CH = 80
NCH = EPW // CH
NPAD = 10240
ROWS_PER_SUB = NPAD // NSUB
DEGW = 8

_mesh = plsc.VectorSubcoreMesh(core_axis_name="c", subcore_axis_name="s")


@functools.partial(
    pl.kernel,
    out_type=jax.ShapeDtypeStruct((NCORE, NPAD, DEGW), jnp.float32),
    mesh=_mesh,
    scratch_types=[
        pltpu.VMEM((NCH, CH), jnp.int32),
        pltpu.VMEM((CH, DEGW), jnp.float32),
        pltpu.VMEM_SHARED((NPAD, DEGW), jnp.float32),
    ],
    compiler_params=pltpu.CompilerParams(use_tc_tiling_on_sc=False),
)
def _deg(dst_hbm, ones_hbm, zeros_hbm, out_hbm, dst_v, ones_v, deg_sh):
    c = lax.axis_index("c")
    s = lax.axis_index("s")
    wid = c * NSUB + s
    pltpu.sync_copy(dst_hbm.at[wid], dst_v)
    pltpu.sync_copy(ones_hbm, ones_v)
    pltpu.sync_copy(zeros_hbm, deg_sh.at[pl.ds(s * ROWS_PER_SUB, ROWS_PER_SUB)])
    plsc.subcore_barrier()

    def body(j, carry):
        pltpu.sync_copy(ones_v, deg_sh.at[dst_v.at[j]], add=True)
        return carry

    lax.fori_loop(0, NCH, body, 0)
    plsc.subcore_barrier()
    pltpu.sync_copy(deg_sh.at[pl.ds(s * ROWS_PER_SUB, ROWS_PER_SUB)],
                    out_hbm.at[c, pl.ds(s * ROWS_PER_SUB, ROWS_PER_SUB)])


@functools.partial(
    pl.kernel,
    out_type=jax.ShapeDtypeStruct((NCORE, NPAD, D), jnp.float32),
    mesh=_mesh,
    scratch_types=[
        pltpu.VMEM((NCH, CH), jnp.int32),
        pltpu.VMEM((NCH, CH), jnp.int32),
        pltpu.VMEM((CH, D), jnp.float32),
        pltpu.VMEM((CH, D), jnp.float32),
        pltpu.VMEM_SHARED((NPAD, D), jnp.float32),
        pltpu.SemaphoreType.DMA,
        pltpu.SemaphoreType.DMA,
    ],
    compiler_params=pltpu.CompilerParams(use_tc_tiling_on_sc=False),
)
def _prop(g_hbm, src_hbm, dst_hbm, zeros_hbm, out_hbm,
          src_v, dst_v, rows0_v, rows1_v, acc_sh, sem0, sem1):
    c = lax.axis_index("c")
    s = lax.axis_index("s")
    wid = c * NSUB + s
    pltpu.sync_copy(src_hbm.at[wid], src_v)
    pltpu.sync_copy(dst_hbm.at[wid], dst_v)
    pltpu.sync_copy(zeros_hbm, acc_sh.at[pl.ds(s * ROWS_PER_SUB, ROWS_PER_SUB)])
    plsc.subcore_barrier()

    pltpu.async_copy(g_hbm.at[src_v.at[0]], rows0_v, sem0)

    def pair(p, carry):
        j0 = p * 2
        j1 = j0 + 1

        pltpu.make_async_copy(g_hbm.at[src_v.at[j0]], rows0_v, sem0).wait()

        @pl.when(j1 < NCH)
        def _():
            pltpu.async_copy(g_hbm.at[src_v.at[j1]], rows1_v, sem1)

        pltpu.sync_copy(rows0_v, acc_sh.at[dst_v.at[j0]], add=True)

        @pl.when(j1 < NCH)
        def _():
            pltpu.make_async_copy(g_hbm.at[src_v.at[j1]], rows1_v, sem1).wait()

            @pl.when(j0 + 2 < NCH)
            def _():
                pltpu.async_copy(g_hbm.at[src_v.at[j0 + 2]], rows0_v, sem0)

            pltpu.sync_copy(rows1_v, acc_sh.at[dst_v.at[j1]], add=True)

        return carry

    lax.fori_loop(0, (NCH + 1) // 2, pair, 0)
    plsc.subcore_barrier()
    pltpu.sync_copy(acc_sh.at[pl.ds(s * ROWS_PER_SUB, ROWS_PER_SUB)],
                    out_hbm.at[c, pl.ds(s * ROWS_PER_SUB, ROWS_PER_SUB)])


def _pre_body(x_ref, w_ref, deg_ref, g_out, dis_out):
    deg = deg_ref[0, 0:N_NODES, 0:1] + deg_ref[1, 0:N_NODES, 0:1] + 1.0
    dis = lax.rsqrt(deg)
    h = jnp.dot(x_ref[...], w_ref[...],
                preferred_element_type=jnp.float32,
                precision=lax.Precision.HIGHEST)
    g_out[...] = h * dis
    dis_out[...] = dis


def _mid_body(acc_ref, g_ref, dis_ref, b_ref, gam_ref, bet_ref, w_ref, gout_ref):
    dis = dis_ref[...]
    sres = ((acc_ref[0, 0:N_NODES, :] + acc_ref[1, 0:N_NODES, :]
            + g_ref[...]) * dis + b_ref[...])
    mu = jnp.mean(sres, axis=0, keepdims=True)
    var = jnp.mean((sres - mu) ** 2, axis=0, keepdims=True)
    h = jnp.maximum((sres - mu) * lax.rsqrt(var + 1e-5) * gam_ref[...]
                    + bet_ref[...], 0.0)
    gout_ref[...] = jnp.dot(h, w_ref[...],
                            preferred_element_type=jnp.float32,
                            precision=lax.Precision.HIGHEST) * dis


def _fin_body(acc_ref, g_ref, dis_ref, b_ref, gam_ref, bet_ref,
              batch_ref, wl_ref, bl_ref, out_ref):
    dis = dis_ref[...]
    sres = ((acc_ref[0, 0:N_NODES, :] + acc_ref[1, 0:N_NODES, :]
            + g_ref[...]) * dis + b_ref[...])
    mu = jnp.mean(sres, axis=0, keepdims=True)
    var = jnp.mean((sres - mu) ** 2, axis=0, keepdims=True)
    h = jnp.maximum((sres - mu) * lax.rsqrt(var + 1e-5) * gam_ref[...]
                    + bet_ref[...], 0.0)
    gids = lax.broadcasted_iota(jnp.int32, (N_GRAPHS, N_NODES), 0)
    mask = jnp.where(gids == batch_ref[...], 1.0, 0.0)
    pooled_sum = jnp.dot(mask, h, preferred_element_type=jnp.float32,
                         precision=lax.Precision.HIGHEST)
    cnt = jnp.sum(mask, axis=1, keepdims=True)
    pooled = pooled_sum / jnp.maximum(cnt, 1.0)
    out_ref[...] = jnp.dot(pooled, wl_ref[...],
                           preferred_element_type=jnp.float32,
                           precision=lax.Precision.HIGHEST) + bl_ref[...]


_pre = pl.pallas_call(
    _pre_body,
    out_shape=[jax.ShapeDtypeStruct((N_NODES, D), jnp.float32),
               jax.ShapeDtypeStruct((N_NODES, 1), jnp.float32)],
)

_mid = pl.pallas_call(
    _mid_body,
    out_shape=jax.ShapeDtypeStruct((N_NODES, D), jnp.float32),
)

_fin = pl.pallas_call(
    _fin_body,
    out_shape=jax.ShapeDtypeStruct((N_GRAPHS, N_CLASSES), jnp.float32),
)


def kernel(x, edge_index, batch, W1, b1, g1, be1, W2, b2, g2, be2,
           W3, b3, g3, be3, Wl, bl):
    src = edge_index[0].astype(jnp.int32).reshape(NW, NCH, CH)
    dst = edge_index[1].astype(jnp.int32).reshape(NW, NCH, CH)
    ones_deg = jnp.ones((CH, DEGW), jnp.float32)
    zeros_deg = jnp.zeros((ROWS_PER_SUB, DEGW), jnp.float32)
    zeros_acc = jnp.zeros((ROWS_PER_SUB, D), jnp.float32)
    b1r, b2r, b3r = b1.reshape(1, D), b2.reshape(1, D), b3.reshape(1, D)
    g1r, g2r, g3r = g1.reshape(1, D), g2.reshape(1, D), g3.reshape(1, D)
    be1r, be2r, be3r = be1.reshape(1, D), be2.reshape(1, D), be3.reshape(1, D)
    batch_r = batch.astype(jnp.int32).reshape(1, N_NODES)

    deg2 = _deg(dst, ones_deg, zeros_deg)
    gl1, dis = _pre(x, W1, deg2)
    acc1 = _prop(gl1, src, dst, zeros_acc)
    gl2 = _mid(acc1, gl1, dis, b1r, g1r, be1r, W2)
    acc2 = _prop(gl2, src, dst, zeros_acc)
    gl3 = _mid(acc2, gl2, dis, b2r, g2r, be2r, W3)
    acc3 = _prop(gl3, src, dst, zeros_acc)
    return _fin(acc3, gl3, dis, b3r, g3r, be3r, batch_r, Wl,
                bl.reshape(1, N_CLASSES))


# speedup vs baseline: 20.9760x; 1.0108x over previous
import functools

import jax
import jax.numpy as jnp
from jax import lax
from jax.experimental import pallas as pl
from jax.experimental.pallas import tpu as pltpu
from jax.experimental.pallas import tpu_sc as plsc

N_NODES = 10000
N_EDGES = 320000
D = 128
N_GRAPHS = 64
N_CLASSES = 10

NCORE = 2
NSUB = 16
NW = NCORE * NSUB
EPW = N_EDGES // NW
CH = 80
NCH = EPW // CH
NPAD = 10240
ROWS_PER_SUB = NPAD // NSUB
DEGW = 8

_mesh = plsc.VectorSubcoreMesh(core_axis_name="c", subcore_axis_name="s")


@functools.partial(
    pl.kernel,
    out_type=jax.ShapeDtypeStruct((NCORE, NPAD, DEGW), jnp.float32),
    mesh=_mesh,
    scratch_types=[
        pltpu.VMEM((NCH, CH), jnp.int32),
        pltpu.VMEM((CH, DEGW), jnp.float32),
        pltpu.VMEM_SHARED((NPAD, DEGW), jnp.float32),
    ],
    compiler_params=pltpu.CompilerParams(use_tc_tiling_on_sc=False),
)
def _deg(dst_hbm, ones_hbm, zeros_hbm, out_hbm, dst_v, ones_v, deg_sh):
    c = lax.axis_index("c")
    s = lax.axis_index("s")
    wid = c * NSUB + s
    pltpu.sync_copy(dst_hbm.at[wid], dst_v)
    pltpu.sync_copy(ones_hbm, ones_v)
    pltpu.sync_copy(zeros_hbm, deg_sh.at[pl.ds(s * ROWS_PER_SUB, ROWS_PER_SUB)])
    plsc.subcore_barrier()

    def body(j, carry):
        pltpu.sync_copy(ones_v, deg_sh.at[dst_v.at[j]], add=True)
        return carry

    lax.fori_loop(0, NCH, body, 0)
    plsc.subcore_barrier()
    pltpu.sync_copy(deg_sh.at[pl.ds(s * ROWS_PER_SUB, ROWS_PER_SUB)],
                    out_hbm.at[c, pl.ds(s * ROWS_PER_SUB, ROWS_PER_SUB)])


@functools.partial(
    pl.kernel,
    out_type=jax.ShapeDtypeStruct((NCORE, NPAD, D), jnp.float32),
    mesh=_mesh,
    scratch_types=[
        pltpu.VMEM((NCH, CH), jnp.int32),
        pltpu.VMEM((NCH, CH), jnp.int32),
        pltpu.VMEM((CH, D), jnp.float32),
        pltpu.VMEM((CH, D), jnp.float32),
        pltpu.VMEM_SHARED((NPAD, D), jnp.float32),
        pltpu.SemaphoreType.DMA,
        pltpu.SemaphoreType.DMA,
        pltpu.SemaphoreType.DMA,
        pltpu.SemaphoreType.DMA,
    ],
    compiler_params=pltpu.CompilerParams(use_tc_tiling_on_sc=False),
)
def _prop(g_hbm, src_hbm, dst_hbm, zeros_hbm, out_hbm,
          src_v, dst_v, rows0_v, rows1_v, acc_sh,
          sem_g0, sem_g1, sem_s0, sem_s1):
    c = lax.axis_index("c")
    s = lax.axis_index("s")
    wid = c * NSUB + s
    pltpu.sync_copy(src_hbm.at[wid], src_v)
    pltpu.sync_copy(dst_hbm.at[wid], dst_v)
    pltpu.sync_copy(zeros_hbm, acc_sh.at[pl.ds(s * ROWS_PER_SUB, ROWS_PER_SUB)])
    plsc.subcore_barrier()

    pltpu.async_copy(g_hbm.at[src_v.at[0]], rows0_v, sem_g0)
    pltpu.async_copy(g_hbm.at[src_v.at[1]], rows1_v, sem_g1)

    def pair(p, carry):
        j0 = p * 2
        j1 = j0 + 1

        pltpu.make_async_copy(g_hbm.at[src_v.at[j0]], rows0_v, sem_g0).wait()
        pltpu.async_copy(rows0_v, acc_sh.at[dst_v.at[j0]], sem_s0, add=True)

        @pl.when(j1 < NCH)
        def _():
            pltpu.make_async_copy(g_hbm.at[src_v.at[j1]], rows1_v, sem_g1).wait()
            pltpu.async_copy(rows1_v, acc_sh.at[dst_v.at[j1]], sem_s1, add=True)

        @pl.when(j0 + 2 < NCH)
        def _():
            pltpu.make_async_copy(rows0_v, acc_sh.at[dst_v.at[j0]], sem_s0).wait()
            pltpu.async_copy(g_hbm.at[src_v.at[j0 + 2]], rows0_v, sem_g0)

        @pl.when(j1 + 2 < NCH)
        def _():
            pltpu.make_async_copy(rows1_v, acc_sh.at[dst_v.at[j1]], sem_s1).wait()
            pltpu.async_copy(g_hbm.at[src_v.at[j1 + 2]], rows1_v, sem_g1)

        return carry

    lax.fori_loop(0, (NCH + 1) // 2, pair, 0)
    pltpu.make_async_copy(rows0_v, acc_sh.at[dst_v.at[NCH - 1]], sem_s0).wait()
    pltpu.make_async_copy(rows1_v, acc_sh.at[dst_v.at[NCH - 2]], sem_s1).wait()
    plsc.subcore_barrier()
    pltpu.sync_copy(acc_sh.at[pl.ds(s * ROWS_PER_SUB, ROWS_PER_SUB)],
                    out_hbm.at[c, pl.ds(s * ROWS_PER_SUB, ROWS_PER_SUB)])


def _pre_body(x_ref, w_ref, deg_ref, g_out, dis_out):
    deg = deg_ref[0, 0:N_NODES, 0:1] + deg_ref[1, 0:N_NODES, 0:1] + 1.0
    dis = lax.rsqrt(deg)
    h = jnp.dot(x_ref[...], w_ref[...],
                preferred_element_type=jnp.float32,
                precision=lax.Precision.HIGHEST)
    g_out[...] = h * dis
    dis_out[...] = dis


def _mid_body(acc_ref, g_ref, dis_ref, b_ref, gam_ref, bet_ref, w_ref, gout_ref):
    dis = dis_ref[...]
    sres = ((acc_ref[0, 0:N_NODES, :] + acc_ref[1, 0:N_NODES, :]
            + g_ref[...]) * dis + b_ref[...])
    mu = jnp.mean(sres, axis=0, keepdims=True)
    var = jnp.mean((sres - mu) ** 2, axis=0, keepdims=True)
    h = jnp.maximum((sres - mu) * lax.rsqrt(var + 1e-5) * gam_ref[...]
                    + bet_ref[...], 0.0)
    gout_ref[...] = jnp.dot(h, w_ref[...],
                            preferred_element_type=jnp.float32,
                            precision=lax.Precision.HIGHEST) * dis


def _fin_body(acc_ref, g_ref, dis_ref, b_ref, gam_ref, bet_ref,
              batch_ref, wl_ref, bl_ref, out_ref):
    dis = dis_ref[...]
    sres = ((acc_ref[0, 0:N_NODES, :] + acc_ref[1, 0:N_NODES, :]
            + g_ref[...]) * dis + b_ref[...])
    mu = jnp.mean(sres, axis=0, keepdims=True)
    var = jnp.mean((sres - mu) ** 2, axis=0, keepdims=True)
    h = jnp.maximum((sres - mu) * lax.rsqrt(var + 1e-5) * gam_ref[...]
                    + bet_ref[...], 0.0)
    gids = lax.broadcasted_iota(jnp.int32, (N_GRAPHS, N_NODES), 0)
    mask = jnp.where(gids == batch_ref[...], 1.0, 0.0)
    pooled_sum = jnp.dot(mask, h, preferred_element_type=jnp.float32,
                         precision=lax.Precision.HIGHEST)
    cnt = jnp.sum(mask, axis=1, keepdims=True)
    pooled = pooled_sum / jnp.maximum(cnt, 1.0)
    out_ref[...] = jnp.dot(pooled, wl_ref[...],
                           preferred_element_type=jnp.float32,
                           precision=lax.Precision.HIGHEST) + bl_ref[...]


_pre = pl.pallas_call(
    _pre_body,
    out_shape=[jax.ShapeDtypeStruct((N_NODES, D), jnp.float32),
               jax.ShapeDtypeStruct((N_NODES, 1), jnp.float32)],
)

_mid = pl.pallas_call(
    _mid_body,
    out_shape=jax.ShapeDtypeStruct((N_NODES, D), jnp.float32),
)

_fin = pl.pallas_call(
    _fin_body,
    out_shape=jax.ShapeDtypeStruct((N_GRAPHS, N_CLASSES), jnp.float32),
)


def kernel(x, edge_index, batch, W1, b1, g1, be1, W2, b2, g2, be2,
           W3, b3, g3, be3, Wl, bl):
    src = edge_index[0].astype(jnp.int32).reshape(NW, NCH, CH)
    dst = edge_index[1].astype(jnp.int32).reshape(NW, NCH, CH)
    ones_deg = jnp.ones((CH, DEGW), jnp.float32)
    zeros_deg = jnp.zeros((ROWS_PER_SUB, DEGW), jnp.float32)
    zeros_acc = jnp.zeros((ROWS_PER_SUB, D), jnp.float32)
    b1r, b2r, b3r = b1.reshape(1, D), b2.reshape(1, D), b3.reshape(1, D)
    g1r, g2r, g3r = g1.reshape(1, D), g2.reshape(1, D), g3.reshape(1, D)
    be1r, be2r, be3r = be1.reshape(1, D), be2.reshape(1, D), be3.reshape(1, D)
    batch_r = batch.astype(jnp.int32).reshape(1, N_NODES)

    deg2 = _deg(dst, ones_deg, zeros_deg)
    gl1, dis = _pre(x, W1, deg2)
    acc1 = _prop(gl1, src, dst, zeros_acc)
    gl2 = _mid(acc1, gl1, dis, b1r, g1r, be1r, W2)
    acc2 = _prop(gl2, src, dst, zeros_acc)
    gl3 = _mid(acc2, gl2, dis, b2r, g2r, be2r, W3)
    acc3 = _prop(gl3, src, dst, zeros_acc)
    return _fin(acc3, gl3, dis, b3r, g3r, be3r, batch_r, Wl,
                bl.reshape(1, N_CLASSES))


# speedup vs baseline: 25.1776x; 1.2003x over previous
import functools

import jax
import jax.numpy as jnp
from jax import lax
from jax.experimental import pallas as pl
from jax.experimental.pallas import tpu as pltpu
from jax.experimental.pallas import tpu_sc as plsc

N_NODES = 10000
N_EDGES = 320000
D = 128
N_GRAPHS = 64
N_CLASSES = 10

NCORE = 2
NSUB = 16
NW = NCORE * NSUB
EPW = N_EDGES // NW
CH = 80
NCH = EPW // CH
NPAD = N_NODES
ROWS_PER_SUB = NPAD // NSUB
DEGW = 8

_mesh = plsc.VectorSubcoreMesh(core_axis_name="c", subcore_axis_name="s")


@functools.partial(
    pl.kernel,
    out_type=jax.ShapeDtypeStruct((NCORE, NPAD, DEGW), jnp.float32),
    mesh=_mesh,
    scratch_types=[
        pltpu.VMEM((NCH, CH), jnp.int32),
        pltpu.VMEM((CH, DEGW), jnp.float32),
        pltpu.VMEM_SHARED((NPAD, DEGW), jnp.float32),
    ],
    compiler_params=pltpu.CompilerParams(use_tc_tiling_on_sc=False),
)
def _deg(dst_hbm, ones_hbm, zeros_hbm, out_hbm, dst_v, ones_v, deg_sh):
    c = lax.axis_index("c")
    s = lax.axis_index("s")
    wid = c * NSUB + s
    pltpu.sync_copy(dst_hbm.at[wid], dst_v)
    pltpu.sync_copy(ones_hbm, ones_v)
    pltpu.sync_copy(zeros_hbm, deg_sh.at[pl.ds(s * ROWS_PER_SUB, ROWS_PER_SUB)])
    plsc.subcore_barrier()

    def body(j, carry):
        pltpu.sync_copy(ones_v, deg_sh.at[dst_v.at[j]], add=True)
        return carry

    lax.fori_loop(0, NCH, body, 0)
    plsc.subcore_barrier()
    pltpu.sync_copy(deg_sh.at[pl.ds(s * ROWS_PER_SUB, ROWS_PER_SUB)],
                    out_hbm.at[c, pl.ds(s * ROWS_PER_SUB, ROWS_PER_SUB)])


@functools.partial(
    pl.kernel,
    out_type=jax.ShapeDtypeStruct((NCORE, NPAD, D), jnp.float32),
    mesh=_mesh,
    scratch_types=[
        pltpu.VMEM((NCH, CH), jnp.int32),
        pltpu.VMEM((NCH, CH), jnp.int32),
        pltpu.VMEM((CH, D), jnp.float32),
        pltpu.VMEM((CH, D), jnp.float32),
        pltpu.VMEM((CH, D), jnp.float32),
        pltpu.VMEM_SHARED((NPAD, D), jnp.float32),
        pltpu.SemaphoreType.DMA,
        pltpu.SemaphoreType.DMA,
        pltpu.SemaphoreType.DMA,
        pltpu.SemaphoreType.DMA,
        pltpu.SemaphoreType.DMA,
        pltpu.SemaphoreType.DMA,
    ],
    compiler_params=pltpu.CompilerParams(use_tc_tiling_on_sc=False),
)
def _prop(g_hbm, src_hbm, dst_hbm, zeros_hbm, out_hbm,
          src_v, dst_v, rows0_v, rows1_v, rows2_v, acc_sh,
          sem_g0, sem_g1, sem_g2, sem_s0, sem_s1, sem_s2):
    c = lax.axis_index("c")
    s = lax.axis_index("s")
    wid = c * NSUB + s
    pltpu.sync_copy(src_hbm.at[wid], src_v)
    pltpu.sync_copy(dst_hbm.at[wid], dst_v)
    pltpu.sync_copy(zeros_hbm, acc_sh.at[pl.ds(s * ROWS_PER_SUB, ROWS_PER_SUB)])
    plsc.subcore_barrier()

    bufs = ((rows0_v, sem_g0, sem_s0),
            (rows1_v, sem_g1, sem_s1),
            (rows2_v, sem_g2, sem_s2))

    def gather(j, b):
        rv, sg, _ = bufs[b]
        pltpu.async_copy(g_hbm.at[src_v.at[j]], rv, sg)

    def wait_gather(j, b):
        rv, sg, _ = bufs[b]
        pltpu.make_async_copy(g_hbm.at[src_v.at[j]], rv, sg).wait()

    def scatter(j, b):
        rv, _, ss = bufs[b]
        pltpu.async_copy(rv, acc_sh.at[dst_v.at[j]], ss, add=True)

    def wait_scatter(j, b):
        rv, _, ss = bufs[b]
        pltpu.make_async_copy(rv, acc_sh.at[dst_v.at[j]], ss).wait()

    gather(0, 0)
    gather(1, 1)
    gather(2, 2)

    NTRIP = NCH // 3

    def trip(t, carry):
        j = t * 3
        for b in range(3):
            wait_gather(j + b, b)
            scatter(j + b, b)
        for b in range(3):
            @pl.when(j + 3 + b < NCH)
            def _(b=b):
                wait_scatter(j + b, b)
                gather(j + 3 + b, b)
        return carry

    lax.fori_loop(0, NTRIP, trip, 0)

    wait_gather(NCH - 2, 0)
    scatter(NCH - 2, 0)
    wait_gather(NCH - 1, 1)
    scatter(NCH - 1, 1)
    wait_scatter(NCH - 3, 2)
    wait_scatter(NCH - 2, 0)
    wait_scatter(NCH - 1, 1)
    plsc.subcore_barrier()
    pltpu.sync_copy(acc_sh.at[pl.ds(s * ROWS_PER_SUB, ROWS_PER_SUB)],
                    out_hbm.at[c, pl.ds(s * ROWS_PER_SUB, ROWS_PER_SUB)])


def _pre_body(x_ref, w_ref, deg_ref, g_out, dis_out):
    deg = deg_ref[0, :, 0:1] + deg_ref[1, :, 0:1] + 1.0
    dis = lax.rsqrt(deg)
    h = jnp.dot(x_ref[...], w_ref[...],
                preferred_element_type=jnp.float32,
                precision=lax.Precision.HIGHEST)
    g_out[...] = h * dis
    dis_out[...] = dis


def _mid_body(acc_ref, g_ref, dis_ref, b_ref, gam_ref, bet_ref, w_ref, gout_ref):
    dis = dis_ref[...]
    sres = (acc_ref[0] + acc_ref[1] + g_ref[...]) * dis + b_ref[...]
    mu = jnp.mean(sres, axis=0, keepdims=True)
    var = jnp.mean((sres - mu) ** 2, axis=0, keepdims=True)
    h = jnp.maximum((sres - mu) * lax.rsqrt(var + 1e-5) * gam_ref[...]
                    + bet_ref[...], 0.0)
    gout_ref[...] = jnp.dot(h, w_ref[...],
                            preferred_element_type=jnp.float32,
                            precision=lax.Precision.HIGHEST) * dis


def _fin_body(acc_ref, g_ref, dis_ref, b_ref, gam_ref, bet_ref,
              batch_ref, wl_ref, bl_ref, out_ref):
    dis = dis_ref[...]
    sres = (acc_ref[0] + acc_ref[1] + g_ref[...]) * dis + b_ref[...]
    mu = jnp.mean(sres, axis=0, keepdims=True)
    var = jnp.mean((sres - mu) ** 2, axis=0, keepdims=True)
    h = jnp.maximum((sres - mu) * lax.rsqrt(var + 1e-5) * gam_ref[...]
                    + bet_ref[...], 0.0)
    gids = lax.broadcasted_iota(jnp.int32, (N_GRAPHS, N_NODES), 0)
    mask = jnp.where(gids == batch_ref[...], 1.0, 0.0)
    pooled_sum = jnp.dot(mask, h, preferred_element_type=jnp.float32,
                         precision=lax.Precision.HIGHEST)
    cnt = jnp.sum(mask, axis=1, keepdims=True)
    pooled = pooled_sum / jnp.maximum(cnt, 1.0)
    out_ref[...] = jnp.dot(pooled, wl_ref[...],
                           preferred_element_type=jnp.float32,
                           precision=lax.Precision.HIGHEST) + bl_ref[...]


_pre = pl.pallas_call(
    _pre_body,
    out_shape=[jax.ShapeDtypeStruct((N_NODES, D), jnp.float32),
               jax.ShapeDtypeStruct((N_NODES, 1), jnp.float32)],
)

_mid = pl.pallas_call(
    _mid_body,
    out_shape=jax.ShapeDtypeStruct((N_NODES, D), jnp.float32),
)

_fin = pl.pallas_call(
    _fin_body,
    out_shape=jax.ShapeDtypeStruct((N_GRAPHS, N_CLASSES), jnp.float32),
)


def kernel(x, edge_index, batch, W1, b1, g1, be1, W2, b2, g2, be2,
           W3, b3, g3, be3, Wl, bl):
    src = edge_index[0].astype(jnp.int32).reshape(NW, NCH, CH)
    dst = edge_index[1].astype(jnp.int32).reshape(NW, NCH, CH)
    ones_deg = jnp.ones((CH, DEGW), jnp.float32)
    zeros_deg = jnp.zeros((ROWS_PER_SUB, DEGW), jnp.float32)
    zeros_acc = jnp.zeros((ROWS_PER_SUB, D), jnp.float32)
    b1r, b2r, b3r = b1.reshape(1, D), b2.reshape(1, D), b3.reshape(1, D)
    g1r, g2r, g3r = g1.reshape(1, D), g2.reshape(1, D), g3.reshape(1, D)
    be1r, be2r, be3r = be1.reshape(1, D), be2.reshape(1, D), be3.reshape(1, D)
    batch_r = batch.astype(jnp.int32).reshape(1, N_NODES)

    deg2 = _deg(dst, ones_deg, zeros_deg)
    gl1, dis = _pre(x, W1, deg2)
    acc1 = _prop(gl1, src, dst, zeros_acc)
    gl2 = _mid(acc1, gl1, dis, b1r, g1r, be1r, W2)
    acc2 = _prop(gl2, src, dst, zeros_acc)
    gl3 = _mid(acc2, gl2, dis, b2r, g2r, be2r, W3)
    acc3 = _prop(gl3, src, dst, zeros_acc)
    return _fin(acc3, gl3, dis, b3r, g3r, be3r, batch_r, Wl,
                bl.reshape(1, N_CLASSES))


# speedup vs baseline: 26.3169x; 1.0453x over previous
import functools

import jax
import jax.numpy as jnp
from jax import lax
from jax.experimental import pallas as pl
from jax.experimental.pallas import tpu as pltpu
from jax.experimental.pallas import tpu_sc as plsc

N_NODES = 10000
N_EDGES = 320000
D = 128
N_GRAPHS = 64
N_CLASSES = 10

NCORE = 2
NSUB = 16
NW = NCORE * NSUB
EPW = N_EDGES // NW
CH = 40
NCH = EPW // CH
NBUF = 6
NPAD = N_NODES
ROWS_PER_SUB = NPAD // NSUB
DEGW = 8

_mesh = plsc.VectorSubcoreMesh(core_axis_name="c", subcore_axis_name="s")


@functools.partial(
    pl.kernel,
    out_type=jax.ShapeDtypeStruct((NCORE, NPAD, DEGW), jnp.float32),
    mesh=_mesh,
    scratch_types=[
        pltpu.VMEM((NCH, CH), jnp.int32),
        pltpu.VMEM((CH, DEGW), jnp.float32),
        pltpu.VMEM_SHARED((NPAD, DEGW), jnp.float32),
    ],
    compiler_params=pltpu.CompilerParams(use_tc_tiling_on_sc=False),
)
def _deg(dst_hbm, ones_hbm, zeros_hbm, out_hbm, dst_v, ones_v, deg_sh):
    c = lax.axis_index("c")
    s = lax.axis_index("s")
    wid = c * NSUB + s
    pltpu.sync_copy(dst_hbm.at[wid], dst_v)
    pltpu.sync_copy(ones_hbm, ones_v)
    pltpu.sync_copy(zeros_hbm, deg_sh.at[pl.ds(s * ROWS_PER_SUB, ROWS_PER_SUB)])
    plsc.subcore_barrier()

    def body(j, carry):
        pltpu.sync_copy(ones_v, deg_sh.at[dst_v.at[j]], add=True)
        return carry

    lax.fori_loop(0, NCH, body, 0)
    plsc.subcore_barrier()
    pltpu.sync_copy(deg_sh.at[pl.ds(s * ROWS_PER_SUB, ROWS_PER_SUB)],
                    out_hbm.at[c, pl.ds(s * ROWS_PER_SUB, ROWS_PER_SUB)])


@functools.partial(
    pl.kernel,
    out_type=jax.ShapeDtypeStruct((NCORE, NPAD, D), jnp.float32),
    mesh=_mesh,
    scratch_types=[
        pltpu.VMEM((NCH, CH), jnp.int32),
        pltpu.VMEM((NCH, CH), jnp.int32),
    ] + [pltpu.VMEM((CH, D), jnp.float32) for _ in range(NBUF)]
      + [pltpu.VMEM_SHARED((NPAD, D), jnp.float32)]
      + [pltpu.SemaphoreType.DMA for _ in range(2 * NBUF)],
    compiler_params=pltpu.CompilerParams(use_tc_tiling_on_sc=False),
)
def _prop(g_hbm, src_hbm, dst_hbm, zeros_hbm, out_hbm,
          src_v, dst_v, *rest):
    rows = rest[:NBUF]
    acc_sh = rest[NBUF]
    sem_g = rest[NBUF + 1:NBUF + 1 + NBUF]
    sem_s = rest[NBUF + 1 + NBUF:]
    c = lax.axis_index("c")
    s = lax.axis_index("s")
    wid = c * NSUB + s
    pltpu.sync_copy(src_hbm.at[wid], src_v)
    pltpu.sync_copy(dst_hbm.at[wid], dst_v)
    pltpu.sync_copy(zeros_hbm, acc_sh.at[pl.ds(s * ROWS_PER_SUB, ROWS_PER_SUB)])
    plsc.subcore_barrier()

    def gather(j, b):
        pltpu.async_copy(g_hbm.at[src_v.at[j]], rows[b], sem_g[b])

    def wait_gather(j, b):
        pltpu.make_async_copy(g_hbm.at[src_v.at[j]], rows[b], sem_g[b]).wait()

    def scatter(j, b):
        pltpu.async_copy(rows[b], acc_sh.at[dst_v.at[j]], sem_s[b], add=True)

    def wait_scatter(j, b):
        pltpu.make_async_copy(rows[b], acc_sh.at[dst_v.at[j]], sem_s[b]).wait()

    for b in range(NBUF):
        gather(b, b)

    NTRIP = NCH // NBUF
    NTAIL = NCH % NBUF

    def trip(t, carry):
        j = t * NBUF
        for b in range(NBUF):
            wait_gather(j + b, b)
            scatter(j + b, b)
        for b in range(NBUF):
            @pl.when(j + NBUF + b < NCH)
            def _(b=b):
                wait_scatter(j + b, b)
                gather(j + NBUF + b, b)
        return carry

    lax.fori_loop(0, NTRIP, trip, 0)

    for b in range(NTAIL):
        wait_gather(NTRIP * NBUF + b, b)
        scatter(NTRIP * NBUF + b, b)
    for b in range(NTAIL, NBUF):
        wait_scatter((NTRIP - 1) * NBUF + b, b)
    for b in range(NTAIL):
        wait_scatter(NTRIP * NBUF + b, b)
    plsc.subcore_barrier()
    pltpu.sync_copy(acc_sh.at[pl.ds(s * ROWS_PER_SUB, ROWS_PER_SUB)],
                    out_hbm.at[c, pl.ds(s * ROWS_PER_SUB, ROWS_PER_SUB)])


def _pre_body(x_ref, w_ref, deg_ref, g_out, dis_out):
    deg = deg_ref[0, :, 0:1] + deg_ref[1, :, 0:1] + 1.0
    dis = lax.rsqrt(deg)
    h = jnp.dot(x_ref[...], w_ref[...],
                preferred_element_type=jnp.float32,
                precision=lax.Precision.HIGHEST)
    g_out[...] = h * dis
    dis_out[...] = dis


def _mid_body(acc_ref, g_ref, dis_ref, b_ref, gam_ref, bet_ref, w_ref, gout_ref):
    dis = dis_ref[...]
    sres = (acc_ref[0] + acc_ref[1] + g_ref[...]) * dis + b_ref[...]
    mu = jnp.mean(sres, axis=0, keepdims=True)
    var = jnp.mean((sres - mu) ** 2, axis=0, keepdims=True)
    h = jnp.maximum((sres - mu) * lax.rsqrt(var + 1e-5) * gam_ref[...]
                    + bet_ref[...], 0.0)
    gout_ref[...] = jnp.dot(h, w_ref[...],
                            preferred_element_type=jnp.float32,
                            precision=lax.Precision.HIGHEST) * dis


def _fin_body(acc_ref, g_ref, dis_ref, b_ref, gam_ref, bet_ref,
              batch_ref, wl_ref, bl_ref, out_ref):
    dis = dis_ref[...]
    sres = (acc_ref[0] + acc_ref[1] + g_ref[...]) * dis + b_ref[...]
    mu = jnp.mean(sres, axis=0, keepdims=True)
    var = jnp.mean((sres - mu) ** 2, axis=0, keepdims=True)
    h = jnp.maximum((sres - mu) * lax.rsqrt(var + 1e-5) * gam_ref[...]
                    + bet_ref[...], 0.0)
    gids = lax.broadcasted_iota(jnp.int32, (N_GRAPHS, N_NODES), 0)
    mask = jnp.where(gids == batch_ref[...], 1.0, 0.0)
    pooled_sum = jnp.dot(mask, h, preferred_element_type=jnp.float32,
                         precision=lax.Precision.HIGHEST)
    cnt = jnp.sum(mask, axis=1, keepdims=True)
    pooled = pooled_sum / jnp.maximum(cnt, 1.0)
    out_ref[...] = jnp.dot(pooled, wl_ref[...],
                           preferred_element_type=jnp.float32,
                           precision=lax.Precision.HIGHEST) + bl_ref[...]


_pre = pl.pallas_call(
    _pre_body,
    out_shape=[jax.ShapeDtypeStruct((N_NODES, D), jnp.float32),
               jax.ShapeDtypeStruct((N_NODES, 1), jnp.float32)],
)

_mid = pl.pallas_call(
    _mid_body,
    out_shape=jax.ShapeDtypeStruct((N_NODES, D), jnp.float32),
)

_fin = pl.pallas_call(
    _fin_body,
    out_shape=jax.ShapeDtypeStruct((N_GRAPHS, N_CLASSES), jnp.float32),
)


def kernel(x, edge_index, batch, W1, b1, g1, be1, W2, b2, g2, be2,
           W3, b3, g3, be3, Wl, bl):
    src = edge_index[0].astype(jnp.int32).reshape(NW, NCH, CH)
    dst = edge_index[1].astype(jnp.int32).reshape(NW, NCH, CH)
    ones_deg = jnp.ones((CH, DEGW), jnp.float32)
    zeros_deg = jnp.zeros((ROWS_PER_SUB, DEGW), jnp.float32)
    zeros_acc = jnp.zeros((ROWS_PER_SUB, D), jnp.float32)
    b1r, b2r, b3r = b1.reshape(1, D), b2.reshape(1, D), b3.reshape(1, D)
    g1r, g2r, g3r = g1.reshape(1, D), g2.reshape(1, D), g3.reshape(1, D)
    be1r, be2r, be3r = be1.reshape(1, D), be2.reshape(1, D), be3.reshape(1, D)
    batch_r = batch.astype(jnp.int32).reshape(1, N_NODES)

    deg2 = _deg(dst, ones_deg, zeros_deg)
    gl1, dis = _pre(x, W1, deg2)
    acc1 = _prop(gl1, src, dst, zeros_acc)
    gl2 = _mid(acc1, gl1, dis, b1r, g1r, be1r, W2)
    acc2 = _prop(gl2, src, dst, zeros_acc)
    gl3 = _mid(acc2, gl2, dis, b2r, g2r, be2r, W3)
    acc3 = _prop(gl3, src, dst, zeros_acc)
    return _fin(acc3, gl3, dis, b3r, g3r, be3r, batch_r, Wl,
                bl.reshape(1, N_CLASSES))


# speedup vs baseline: 26.4621x; 1.0055x over previous
import functools

import jax
import jax.numpy as jnp
from jax import lax
from jax.experimental import pallas as pl
from jax.experimental.pallas import tpu as pltpu
from jax.experimental.pallas import tpu_sc as plsc

N_NODES = 10000
N_EDGES = 320000
D = 128
N_GRAPHS = 64
N_CLASSES = 10

NCORE = 2
NSUB = 16
NW = NCORE * NSUB
EPW = N_EDGES // NW
CH = 40
NCH = EPW // CH
NBUF = 6
NPAD = N_NODES
ROWS_PER_SUB = NPAD // NSUB
DEGW = 8

_mesh = plsc.VectorSubcoreMesh(core_axis_name="c", subcore_axis_name="s")


@functools.partial(
    pl.kernel,
    out_type=jax.ShapeDtypeStruct((NCORE, NPAD, DEGW), jnp.float32),
    mesh=_mesh,
    scratch_types=[
        pltpu.VMEM((NCH, CH), jnp.int32),
        pltpu.VMEM((CH, DEGW), jnp.float32),
        pltpu.VMEM_SHARED((NPAD, DEGW), jnp.float32),
    ],
    compiler_params=pltpu.CompilerParams(use_tc_tiling_on_sc=False),
)
def _deg(dst_hbm, ones_hbm, zeros_hbm, out_hbm, dst_v, ones_v, deg_sh):
    c = lax.axis_index("c")
    s = lax.axis_index("s")
    wid = c * NSUB + s
    pltpu.sync_copy(dst_hbm.at[wid], dst_v)
    pltpu.sync_copy(ones_hbm, ones_v)
    pltpu.sync_copy(zeros_hbm, deg_sh.at[pl.ds(s * ROWS_PER_SUB, ROWS_PER_SUB)])
    plsc.subcore_barrier()

    def body(j, carry):
        pltpu.sync_copy(ones_v, deg_sh.at[dst_v.at[j]], add=True)
        return carry

    lax.fori_loop(0, NCH, body, 0)
    plsc.subcore_barrier()
    pltpu.sync_copy(deg_sh.at[pl.ds(s * ROWS_PER_SUB, ROWS_PER_SUB)],
                    out_hbm.at[c, pl.ds(s * ROWS_PER_SUB, ROWS_PER_SUB)])


@functools.partial(
    pl.kernel,
    out_type=jax.ShapeDtypeStruct((NCORE, NPAD, D), jnp.float32),
    mesh=_mesh,
    scratch_types=[
        pltpu.VMEM((NCH, CH), jnp.int32),
        pltpu.VMEM((NCH, CH), jnp.int32),
    ] + [pltpu.VMEM((CH, D), jnp.float32) for _ in range(NBUF)]
      + [pltpu.VMEM_SHARED((NPAD, D), jnp.float32)]
      + [pltpu.SemaphoreType.DMA for _ in range(2 * NBUF)],
    compiler_params=pltpu.CompilerParams(use_tc_tiling_on_sc=False),
)
def _prop(g_hbm, src_hbm, dst_hbm, zeros_hbm, out_hbm,
          src_v, dst_v, *rest):
    rows = rest[:NBUF]
    acc_sh = rest[NBUF]
    sem_g = rest[NBUF + 1:NBUF + 1 + NBUF]
    sem_s = rest[NBUF + 1 + NBUF:]
    c = lax.axis_index("c")
    s = lax.axis_index("s")
    wid = c * NSUB + s
    pltpu.sync_copy(src_hbm.at[wid], src_v)
    pltpu.sync_copy(dst_hbm.at[wid], dst_v)
    pltpu.sync_copy(zeros_hbm, acc_sh.at[pl.ds(s * ROWS_PER_SUB, ROWS_PER_SUB)])
    plsc.subcore_barrier()

    def gather(j, b):
        pltpu.async_copy(g_hbm.at[src_v.at[j]], rows[b], sem_g[b])

    def wait_gather(j, b):
        pltpu.make_async_copy(g_hbm.at[src_v.at[j]], rows[b], sem_g[b]).wait()

    def scatter(j, b):
        pltpu.async_copy(rows[b], acc_sh.at[dst_v.at[j]], sem_s[b], add=True)

    def wait_scatter(j, b):
        pltpu.make_async_copy(rows[b], acc_sh.at[dst_v.at[j]], sem_s[b]).wait()

    for b in range(NBUF):
        gather(b, b)

    NTRIP = NCH // NBUF
    NTAIL = NCH % NBUF

    def trip(t, carry):
        j = t * NBUF
        for b in range(NBUF):
            wait_gather(j + b, b)
            scatter(j + b, b)
        for b in range(NBUF):
            @pl.when(j + NBUF + b < NCH)
            def _(b=b):
                wait_scatter(j + b, b)
                gather(j + NBUF + b, b)
        return carry

    lax.fori_loop(0, NTRIP, trip, 0)

    for b in range(NTAIL):
        wait_gather(NTRIP * NBUF + b, b)
        scatter(NTRIP * NBUF + b, b)
    for b in range(NTAIL, NBUF):
        wait_scatter((NTRIP - 1) * NBUF + b, b)
    for b in range(NTAIL):
        wait_scatter(NTRIP * NBUF + b, b)
    plsc.subcore_barrier()
    pltpu.sync_copy(acc_sh.at[pl.ds(s * ROWS_PER_SUB, ROWS_PER_SUB)],
                    out_hbm.at[c, pl.ds(s * ROWS_PER_SUB, ROWS_PER_SUB)])


def _mm_body(x_ref, w_ref, u_out):
    u_out[...] = jnp.dot(x_ref[...], w_ref[...],
                         preferred_element_type=jnp.float32,
                         precision=lax.Precision.HIGHEST)


def _scale_body(u_ref, deg_ref, g_out, dis_out):
    deg = deg_ref[0, :, 0:1] + deg_ref[1, :, 0:1] + 1.0
    dis = lax.rsqrt(deg)
    g_out[...] = u_ref[...] * dis
    dis_out[...] = dis


def _mid_body(acc_ref, g_ref, dis_ref, b_ref, gam_ref, bet_ref, w_ref, gout_ref):
    dis = dis_ref[...]
    sres = (acc_ref[0] + acc_ref[1] + g_ref[...]) * dis + b_ref[...]
    mu = jnp.mean(sres, axis=0, keepdims=True)
    var = jnp.mean((sres - mu) ** 2, axis=0, keepdims=True)
    h = jnp.maximum((sres - mu) * lax.rsqrt(var + 1e-5) * gam_ref[...]
                    + bet_ref[...], 0.0)
    gout_ref[...] = jnp.dot(h, w_ref[...],
                            preferred_element_type=jnp.float32,
                            precision=lax.Precision.HIGHEST) * dis


def _fin_body(acc_ref, g_ref, dis_ref, b_ref, gam_ref, bet_ref,
              batch_ref, wl_ref, bl_ref, out_ref):
    dis = dis_ref[...]
    sres = (acc_ref[0] + acc_ref[1] + g_ref[...]) * dis + b_ref[...]
    mu = jnp.mean(sres, axis=0, keepdims=True)
    var = jnp.mean((sres - mu) ** 2, axis=0, keepdims=True)
    h = jnp.maximum((sres - mu) * lax.rsqrt(var + 1e-5) * gam_ref[...]
                    + bet_ref[...], 0.0)
    gids = lax.broadcasted_iota(jnp.int32, (N_GRAPHS, N_NODES), 0)
    mask = jnp.where(gids == batch_ref[...], 1.0, 0.0)
    pooled_sum = jnp.dot(mask, h, preferred_element_type=jnp.float32,
                         precision=lax.Precision.HIGHEST)
    cnt = jnp.sum(mask, axis=1, keepdims=True)
    pooled = pooled_sum / jnp.maximum(cnt, 1.0)
    out_ref[...] = jnp.dot(pooled, wl_ref[...],
                           preferred_element_type=jnp.float32,
                           precision=lax.Precision.HIGHEST) + bl_ref[...]


_mm = pl.pallas_call(
    _mm_body,
    out_shape=jax.ShapeDtypeStruct((N_NODES, D), jnp.float32),
)

_scale = pl.pallas_call(
    _scale_body,
    out_shape=[jax.ShapeDtypeStruct((N_NODES, D), jnp.float32),
               jax.ShapeDtypeStruct((N_NODES, 1), jnp.float32)],
)

_mid = pl.pallas_call(
    _mid_body,
    out_shape=jax.ShapeDtypeStruct((N_NODES, D), jnp.float32),
)

_fin = pl.pallas_call(
    _fin_body,
    out_shape=jax.ShapeDtypeStruct((N_GRAPHS, N_CLASSES), jnp.float32),
)


def kernel(x, edge_index, batch, W1, b1, g1, be1, W2, b2, g2, be2,
           W3, b3, g3, be3, Wl, bl):
    src = edge_index[0].astype(jnp.int32).reshape(NW, NCH, CH)
    dst = edge_index[1].astype(jnp.int32).reshape(NW, NCH, CH)
    ones_deg = jnp.ones((CH, DEGW), jnp.float32)
    zeros_deg = jnp.zeros((ROWS_PER_SUB, DEGW), jnp.float32)
    zeros_acc = jnp.zeros((ROWS_PER_SUB, D), jnp.float32)
    b1r, b2r, b3r = b1.reshape(1, D), b2.reshape(1, D), b3.reshape(1, D)
    g1r, g2r, g3r = g1.reshape(1, D), g2.reshape(1, D), g3.reshape(1, D)
    be1r, be2r, be3r = be1.reshape(1, D), be2.reshape(1, D), be3.reshape(1, D)
    batch_r = batch.astype(jnp.int32).reshape(1, N_NODES)

    deg2 = _deg(dst, ones_deg, zeros_deg)
    u1 = _mm(x, W1)
    gl1, dis = _scale(u1, deg2)
    acc1 = _prop(gl1, src, dst, zeros_acc)
    gl2 = _mid(acc1, gl1, dis, b1r, g1r, be1r, W2)
    acc2 = _prop(gl2, src, dst, zeros_acc)
    gl3 = _mid(acc2, gl2, dis, b2r, g2r, be2r, W3)
    acc3 = _prop(gl3, src, dst, zeros_acc)
    return _fin(acc3, gl3, dis, b3r, g3r, be3r, batch_r, Wl,
                bl.reshape(1, N_CLASSES))


# speedup vs baseline: 26.6706x; 1.0079x over previous
import functools

import jax
import jax.numpy as jnp
from jax import lax
from jax.experimental import pallas as pl
from jax.experimental.pallas import tpu as pltpu
from jax.experimental.pallas import tpu_sc as plsc

N_NODES = 10000
N_EDGES = 320000
D = 128
N_GRAPHS = 64
N_CLASSES = 10

NCORE = 2
NSUB = 16
NW = NCORE * NSUB
EPW = N_EDGES // NW
CH = 40
NCH = EPW // CH
NBUF = 6
NPAD = N_NODES
ROWS_PER_SUB = NPAD // NSUB
DEGW = 8

_mesh = plsc.VectorSubcoreMesh(core_axis_name="c", subcore_axis_name="s")


@functools.partial(
    pl.kernel,
    out_type=jax.ShapeDtypeStruct((NCORE, NPAD, DEGW), jnp.float32),
    mesh=_mesh,
    scratch_types=[
        pltpu.VMEM((NCH, CH), jnp.int32),
        pltpu.VMEM((CH, DEGW), jnp.float32),
        pltpu.VMEM_SHARED((NPAD, DEGW), jnp.float32),
    ],
    compiler_params=pltpu.CompilerParams(use_tc_tiling_on_sc=False),
)
def _deg(dst_hbm, ones_hbm, zeros_hbm, out_hbm, dst_v, ones_v, deg_sh):
    c = lax.axis_index("c")
    s = lax.axis_index("s")
    wid = c * NSUB + s
    pltpu.sync_copy(dst_hbm.at[wid], dst_v)
    pltpu.sync_copy(ones_hbm, ones_v)
    pltpu.sync_copy(zeros_hbm, deg_sh.at[pl.ds(s * ROWS_PER_SUB, ROWS_PER_SUB)])
    plsc.subcore_barrier()

    def body(j, carry):
        pltpu.sync_copy(ones_v, deg_sh.at[dst_v.at[j]], add=True)
        return carry

    lax.fori_loop(0, NCH, body, 0)
    plsc.subcore_barrier()
    pltpu.sync_copy(deg_sh.at[pl.ds(s * ROWS_PER_SUB, ROWS_PER_SUB)],
                    out_hbm.at[c, pl.ds(s * ROWS_PER_SUB, ROWS_PER_SUB)])


@functools.partial(
    pl.kernel,
    out_type=jax.ShapeDtypeStruct((NCORE, NPAD, D), jnp.float32),
    mesh=_mesh,
    scratch_types=[
        pltpu.VMEM((NCH, CH), jnp.int32),
        pltpu.VMEM((NCH, CH), jnp.int32),
    ] + [pltpu.VMEM((CH, D), jnp.float32) for _ in range(NBUF)]
      + [pltpu.VMEM_SHARED((NPAD, D), jnp.float32)]
      + [pltpu.SemaphoreType.DMA for _ in range(2 * NBUF)],
    compiler_params=pltpu.CompilerParams(use_tc_tiling_on_sc=False),
)
def _prop(g_hbm, src_hbm, dst_hbm, zeros_hbm, out_hbm,
          src_v, dst_v, *rest):
    rows = rest[:NBUF]
    acc_sh = rest[NBUF]
    sem_g = rest[NBUF + 1:NBUF + 1 + NBUF]
    sem_s = rest[NBUF + 1 + NBUF:]
    c = lax.axis_index("c")
    s = lax.axis_index("s")
    wid = c * NSUB + s
    cp_src = pltpu.async_copy(src_hbm.at[wid], src_v, sem_g[0])
    cp_dst = pltpu.async_copy(dst_hbm.at[wid], dst_v, sem_g[1])
    cp_zero = pltpu.async_copy(
        zeros_hbm, acc_sh.at[pl.ds(s * ROWS_PER_SUB, ROWS_PER_SUB)], sem_g[2])
    cp_src.wait()
    cp_dst.wait()
    cp_zero.wait()
    plsc.subcore_barrier()

    def gather(j, b):
        pltpu.async_copy(g_hbm.at[src_v.at[j]], rows[b], sem_g[b])

    def wait_gather(j, b):
        pltpu.make_async_copy(g_hbm.at[src_v.at[j]], rows[b], sem_g[b]).wait()

    def scatter(j, b):
        pltpu.async_copy(rows[b], acc_sh.at[dst_v.at[j]], sem_s[b], add=True)

    def wait_scatter(j, b):
        pltpu.make_async_copy(rows[b], acc_sh.at[dst_v.at[j]], sem_s[b]).wait()

    for b in range(NBUF):
        gather(b, b)

    NTRIP = NCH // NBUF
    NTAIL = NCH % NBUF

    def trip(t, carry):
        j = t * NBUF
        for b in range(NBUF):
            wait_gather(j + b, b)
            scatter(j + b, b)
        for b in range(NBUF):
            @pl.when(j + NBUF + b < NCH)
            def _(b=b):
                wait_scatter(j + b, b)
                gather(j + NBUF + b, b)
        return carry

    lax.fori_loop(0, NTRIP, trip, 0)

    for b in range(NTAIL):
        wait_gather(NTRIP * NBUF + b, b)
        scatter(NTRIP * NBUF + b, b)
    for b in range(NTAIL, NBUF):
        wait_scatter((NTRIP - 1) * NBUF + b, b)
    for b in range(NTAIL):
        wait_scatter(NTRIP * NBUF + b, b)
    plsc.subcore_barrier()
    pltpu.sync_copy(acc_sh.at[pl.ds(s * ROWS_PER_SUB, ROWS_PER_SUB)],
                    out_hbm.at[c, pl.ds(s * ROWS_PER_SUB, ROWS_PER_SUB)])


def _mm_body(x_ref, w_ref, u_out):
    u_out[...] = jnp.dot(x_ref[...], w_ref[...],
                         preferred_element_type=jnp.float32,
                         precision=lax.Precision.HIGHEST)


def _scale_body(u_ref, deg_ref, g_out, dis_out):
    deg = deg_ref[0, :, 0:1] + deg_ref[1, :, 0:1] + 1.0
    dis = lax.rsqrt(deg)
    g_out[...] = u_ref[...] * dis
    dis_out[...] = dis


def _mid_body(acc_ref, g_ref, dis_ref, b_ref, gam_ref, bet_ref, w_ref, gout_ref):
    dis = dis_ref[...]
    sres = (acc_ref[0] + acc_ref[1] + g_ref[...]) * dis + b_ref[...]
    mu = jnp.mean(sres, axis=0, keepdims=True)
    var = jnp.mean((sres - mu) ** 2, axis=0, keepdims=True)
    h = jnp.maximum((sres - mu) * lax.rsqrt(var + 1e-5) * gam_ref[...]
                    + bet_ref[...], 0.0)
    gout_ref[...] = jnp.dot(h, w_ref[...],
                            preferred_element_type=jnp.float32,
                            precision=lax.Precision.HIGHEST) * dis


def _fin_body(acc_ref, g_ref, dis_ref, b_ref, gam_ref, bet_ref,
              batch_ref, wl_ref, bl_ref, out_ref):
    dis = dis_ref[...]
    sres = (acc_ref[0] + acc_ref[1] + g_ref[...]) * dis + b_ref[...]
    mu = jnp.mean(sres, axis=0, keepdims=True)
    var = jnp.mean((sres - mu) ** 2, axis=0, keepdims=True)
    h = jnp.maximum((sres - mu) * lax.rsqrt(var + 1e-5) * gam_ref[...]
                    + bet_ref[...], 0.0)
    gids = lax.broadcasted_iota(jnp.int32, (N_GRAPHS, N_NODES), 0)
    mask = jnp.where(gids == batch_ref[...], 1.0, 0.0)
    pooled_sum = jnp.dot(mask, h, preferred_element_type=jnp.float32,
                         precision=lax.Precision.HIGHEST)
    cnt = jnp.sum(mask, axis=1, keepdims=True)
    pooled = pooled_sum / jnp.maximum(cnt, 1.0)
    out_ref[...] = jnp.dot(pooled, wl_ref[...],
                           preferred_element_type=jnp.float32,
                           precision=lax.Precision.HIGHEST) + bl_ref[...]


_mm = pl.pallas_call(
    _mm_body,
    out_shape=jax.ShapeDtypeStruct((N_NODES, D), jnp.float32),
)

_scale = pl.pallas_call(
    _scale_body,
    out_shape=[jax.ShapeDtypeStruct((N_NODES, D), jnp.float32),
               jax.ShapeDtypeStruct((N_NODES, 1), jnp.float32)],
)

_mid = pl.pallas_call(
    _mid_body,
    out_shape=jax.ShapeDtypeStruct((N_NODES, D), jnp.float32),
)

_fin = pl.pallas_call(
    _fin_body,
    out_shape=jax.ShapeDtypeStruct((N_GRAPHS, N_CLASSES), jnp.float32),
)


def kernel(x, edge_index, batch, W1, b1, g1, be1, W2, b2, g2, be2,
           W3, b3, g3, be3, Wl, bl):
    src = edge_index[0].astype(jnp.int32).reshape(NW, NCH, CH)
    dst = edge_index[1].astype(jnp.int32).reshape(NW, NCH, CH)
    ones_deg = jnp.ones((CH, DEGW), jnp.float32)
    zeros_deg = jnp.zeros((ROWS_PER_SUB, DEGW), jnp.float32)
    zeros_acc = jnp.zeros((ROWS_PER_SUB, D), jnp.float32)
    b1r, b2r, b3r = b1.reshape(1, D), b2.reshape(1, D), b3.reshape(1, D)
    g1r, g2r, g3r = g1.reshape(1, D), g2.reshape(1, D), g3.reshape(1, D)
    be1r, be2r, be3r = be1.reshape(1, D), be2.reshape(1, D), be3.reshape(1, D)
    batch_r = batch.astype(jnp.int32).reshape(1, N_NODES)

    deg2 = _deg(dst, ones_deg, zeros_deg)
    u1 = _mm(x, W1)
    gl1, dis = _scale(u1, deg2)
    acc1 = _prop(gl1, src, dst, zeros_acc)
    gl2 = _mid(acc1, gl1, dis, b1r, g1r, be1r, W2)
    acc2 = _prop(gl2, src, dst, zeros_acc)
    gl3 = _mid(acc2, gl2, dis, b2r, g2r, be2r, W3)
    acc3 = _prop(gl3, src, dst, zeros_acc)
    return _fin(acc3, gl3, dis, b3r, g3r, be3r, batch_r, Wl,
                bl.reshape(1, N_CLASSES))
